# baseline (speedup 1.0000x reference)
import functools

import jax
import jax.numpy as jnp
from jax import lax
from jax.experimental import pallas as pl
from jax.experimental.pallas import tpu as pltpu
from jax.experimental.pallas import tpu_sc as plsc

BATCH = 4096
SEQ = 200
HIDDEN = 64
N = BATCH * SEQ
HEAD = 128
TAIL = SEQ - HEAD

_info = plsc.get_sparse_core_info()
NC, NS = _info.num_cores, _info.num_subcores
NW = NC * NS

ROWS_PER_W = BATCH // NW
NBUF = 4

_mesh = plsc.VectorSubcoreMesh(core_axis_name="c", subcore_axis_name="s")


@functools.partial(
    pl.kernel,
    out_type=jax.ShapeDtypeStruct((2 * BATCH * HEAD,), jnp.int32),
    mesh=_mesh,
    scratch_types=[
        pltpu.VMEM((ROWS_PER_W, HEAD), jnp.int32),
        pltpu.VMEM((ROWS_PER_W, HEAD), jnp.int32),
        pltpu.SemaphoreType.DMA,
        pltpu.SemaphoreType.DMA,
    ],
)
def _repack_ids(ids_hbm, out_hbm, hbuf, tbuf, sem_in, sem_out):
    wid = lax.axis_index("s") * NC + lax.axis_index("c")
    row0 = wid * ROWS_PER_W
    pltpu.make_async_copy(
        ids_hbm.at[pl.ds(row0, ROWS_PER_W), pl.ds(0, HEAD)], hbuf,
        sem_in).start()
    pltpu.make_async_copy(
        ids_hbm.at[pl.ds(row0, ROWS_PER_W), pl.ds(HEAD, HEAD)], tbuf,
        sem_in).start()
    pltpu.make_async_copy(
        ids_hbm.at[pl.ds(row0, ROWS_PER_W), pl.ds(0, HEAD)], hbuf,
        sem_in).wait()
    pltpu.make_async_copy(
        ids_hbm.at[pl.ds(row0, ROWS_PER_W), pl.ds(HEAD, HEAD)], tbuf,
        sem_in).wait()
    cps = []
    for r in range(ROWS_PER_W):
        cps.append(pltpu.make_async_copy(
            hbuf.at[r], out_hbm.at[pl.ds((row0 + r) * HEAD, HEAD)],
            sem_out))
        cps.append(pltpu.make_async_copy(
            tbuf.at[r],
            out_hbm.at[pl.ds((BATCH + row0 + r) * HEAD, HEAD)], sem_out))
    for c in cps:
        c.start()
    for c in cps:
        c.wait()


@functools.partial(
    pl.kernel,
    out_type=jax.ShapeDtypeStruct((BATCH, SEQ, HIDDEN), jnp.float32),
    mesh=_mesh,
    scratch_types=[
        pltpu.VMEM((NBUF, 2 * HEAD), jnp.int32),
        pltpu.VMEM((NBUF, SEQ, HIDDEN), jnp.float32),
        pltpu.SemaphoreType.DMA((NBUF,)),
        pltpu.SemaphoreType.DMA((NBUF,)),
        pltpu.SemaphoreType.DMA((NBUF,)),
    ],
    compiler_params=pltpu.CompilerParams(use_tc_tiling_on_sc=False),
)
def _gather_kernel(ids_hbm, tab_hbm, out_hbm, idx_v, rows_v, sem_i, sem_g,
                   sem_o):
    wid = lax.axis_index("s") * NC + lax.axis_index("c")
    base = wid * ROWS_PER_W

    def idx_copies(c, slot):
        return [
            pltpu.make_async_copy(
                ids_hbm.at[pl.ds((base + c) * HEAD, HEAD)],
                idx_v.at[slot, pl.ds(0, HEAD)], sem_i.at[slot]),
            pltpu.make_async_copy(
                ids_hbm.at[pl.ds((BATCH + base + c) * HEAD, HEAD)],
                idx_v.at[slot, pl.ds(HEAD, HEAD)], sem_i.at[slot]),
        ]

    def gathers(c, slot):
        return [
            pltpu.make_async_copy(
                tab_hbm.at[idx_v.at[slot, pl.ds(0, HEAD)]],
                rows_v.at[slot, pl.ds(0, HEAD), :], sem_g.at[slot]),
            pltpu.make_async_copy(
                tab_hbm.at[idx_v.at[slot, pl.ds(HEAD, TAIL)]],
                rows_v.at[slot, pl.ds(HEAD, TAIL), :], sem_g.at[slot]),
        ]

    def out_copy(c, slot):
        return pltpu.make_async_copy(
            rows_v.at[slot], out_hbm.at[base + c], sem_o.at[slot])

    for b in range(NBUF):
        for cp in idx_copies(b, b):
            cp.start()

    def group(g, carry):
        for b in range(NBUF):
            c = g * NBUF + b
            for cp in idx_copies(c, b):
                cp.wait()
            pl.when(g > 0)(lambda b=b: out_copy(0, b).wait())
            for cp in gathers(c, b):
                cp.start()
            bp = (b - 1) % NBUF

            def finish(c=c, b=b, bp=bp):
                for cp in gathers(c - 1, bp):
                    cp.wait()
                out_copy(c - 1, bp).start()

                def prefetch(c=c, bp=bp):
                    for cp in idx_copies(c + NBUF - 1, bp):
                        cp.start()

                pl.when(c + NBUF - 1 < ROWS_PER_W)(prefetch)

            if b == 0:
                pl.when(g > 0)(finish)
            else:
                finish()
        return carry

    lax.fori_loop(0, ROWS_PER_W // NBUF, group, 0, unroll=False)

    bl = (ROWS_PER_W - 1) % NBUF
    for cp in gathers(ROWS_PER_W - 1, bl):
        cp.wait()
    out_copy(ROWS_PER_W - 1, bl).start()
    for b in range(NBUF):
        out_copy(0, b).wait()


def kernel(input_ids, word_embeddings):
    ids2 = jnp.pad(input_ids.astype(jnp.int32), ((0, 0), (0, 2 * HEAD - SEQ)))
    packed = _repack_ids(ids2)
    return _gather_kernel(packed, word_embeddings)


# speedup vs baseline: 1.0566x; 1.0566x over previous
import functools

import jax
import jax.numpy as jnp
from jax import lax
from jax.experimental import pallas as pl
from jax.experimental.pallas import tpu as pltpu
from jax.experimental.pallas import tpu_sc as plsc

BATCH = 4096
SEQ = 200
HIDDEN = 64
VOCAB = 1000000
N = BATCH * SEQ
HEAD = 128
TAIL = SEQ - HEAD

_info = plsc.get_sparse_core_info()
NC, NS = _info.num_cores, _info.num_subcores
NW = NC * NS

ROWS_PER_W = BATCH // NW
NBUF = 4

_mesh = plsc.VectorSubcoreMesh(core_axis_name="c", subcore_axis_name="s")


@functools.partial(
    pl.kernel,
    out_type=jax.ShapeDtypeStruct((2 * BATCH * HEAD,), jnp.int32),
    mesh=_mesh,
    scratch_types=[
        pltpu.VMEM((ROWS_PER_W, HEAD), jnp.int32),
        pltpu.VMEM((ROWS_PER_W, HEAD), jnp.int32),
        pltpu.SemaphoreType.DMA,
        pltpu.SemaphoreType.DMA,
    ],
)
def _repack_ids(ids_hbm, out_hbm, hbuf, tbuf, sem_in, sem_out):
    wid = lax.axis_index("s") * NC + lax.axis_index("c")
    row0 = wid * ROWS_PER_W
    pltpu.make_async_copy(
        ids_hbm.at[pl.ds(row0, ROWS_PER_W), pl.ds(0, HEAD)], hbuf,
        sem_in).start()
    pltpu.make_async_copy(
        ids_hbm.at[pl.ds(row0, ROWS_PER_W), pl.ds(HEAD, HEAD)], tbuf,
        sem_in).start()
    pltpu.make_async_copy(
        ids_hbm.at[pl.ds(row0, ROWS_PER_W), pl.ds(0, HEAD)], hbuf,
        sem_in).wait()
    pltpu.make_async_copy(
        ids_hbm.at[pl.ds(row0, ROWS_PER_W), pl.ds(HEAD, HEAD)], tbuf,
        sem_in).wait()
    cps = []
    for r in range(ROWS_PER_W):
        cps.append(pltpu.make_async_copy(
            hbuf.at[r], out_hbm.at[pl.ds((row0 + r) * HEAD, HEAD)],
            sem_out))
        cps.append(pltpu.make_async_copy(
            tbuf.at[r],
            out_hbm.at[pl.ds((BATCH + row0 + r) * HEAD, HEAD)], sem_out))
    for c in cps:
        c.start()
    for c in cps:
        c.wait()


@functools.partial(
    pl.kernel,
    out_type=jax.ShapeDtypeStruct((BATCH, SEQ, HIDDEN), jnp.float32),
    mesh=_mesh,
    scratch_types=[
        pltpu.VMEM((NBUF, 2 * HEAD), jnp.int32),
        pltpu.VMEM((NBUF, SEQ, HIDDEN), jnp.float32),
        pltpu.SemaphoreType.DMA((NBUF,)),
        pltpu.SemaphoreType.DMA((NBUF,)),
        pltpu.SemaphoreType.DMA((NBUF,)),
    ],
    compiler_params=pltpu.CompilerParams(use_tc_tiling_on_sc=False),
)
def _gather_kernel(ids_hbm, tab_hbm, out_hbm, idx_v, rows_v, sem_i, sem_g,
                   sem_o):
    wid = lax.axis_index("s") * NC + lax.axis_index("c")
    base = wid * ROWS_PER_W

    def idx_copies(c, slot):
        return [
            pltpu.make_async_copy(
                ids_hbm.at[pl.ds((base + c) * HEAD, HEAD)],
                idx_v.at[slot, pl.ds(0, HEAD)], sem_i.at[slot]),
            pltpu.make_async_copy(
                ids_hbm.at[pl.ds((BATCH + base + c) * HEAD, HEAD)],
                idx_v.at[slot, pl.ds(HEAD, HEAD)], sem_i.at[slot]),
        ]

    def gathers(c, slot):
        return [
            pltpu.make_async_copy(
                tab_hbm.at[idx_v.at[slot, pl.ds(0, HEAD)]],
                rows_v.at[slot, pl.ds(0, HEAD), :], sem_g.at[slot]),
            pltpu.make_async_copy(
                tab_hbm.at[idx_v.at[slot, pl.ds(HEAD, TAIL)]],
                rows_v.at[slot, pl.ds(HEAD, TAIL), :], sem_g.at[slot]),
        ]

    def out_copy(c, slot):
        return pltpu.make_async_copy(
            rows_v.at[slot], out_hbm.at[base + c], sem_o.at[slot])

    for b in range(NBUF):
        for cp in idx_copies(b, b):
            cp.start()

    def group(g, carry):
        for b in range(NBUF):
            c = g * NBUF + b
            for cp in idx_copies(c, b):
                cp.wait()
            pl.when(g > 0)(lambda b=b: out_copy(0, b).wait())
            for cp in gathers(c, b):
                cp.start()
            bp = (b - 1) % NBUF

            def finish(c=c, b=b, bp=bp):
                for cp in gathers(c - 1, bp):
                    cp.wait()
                out_copy(c - 1, bp).start()

                def prefetch(c=c, bp=bp):
                    for cp in idx_copies(c + NBUF - 1, bp):
                        cp.start()

                pl.when(c + NBUF - 1 < ROWS_PER_W)(prefetch)

            if b == 0:
                pl.when(g > 0)(finish)
            else:
                finish()
        return carry

    lax.fori_loop(0, ROWS_PER_W // NBUF, group, 0, unroll=False)

    bl = (ROWS_PER_W - 1) % NBUF
    for cp in gathers(ROWS_PER_W - 1, bl):
        cp.wait()
    out_copy(ROWS_PER_W - 1, bl).start()
    for b in range(NBUF):
        out_copy(0, b).wait()


def kernel(input_ids, word_embeddings):
    ids2 = jnp.pad(input_ids.astype(jnp.int32) * 2,
                   ((0, 0), (0, 2 * HEAD - SEQ)))
    packed = _repack_ids(ids2)
    tab2 = jnp.pad(word_embeddings, ((0, 0), (0, HIDDEN))
                   ).reshape(2 * VOCAB, HIDDEN)
    return _gather_kernel(packed, tab2)


# speedup vs baseline: 1.4240x; 1.3476x over previous
import functools

import jax
import jax.numpy as jnp
from jax import lax
from jax.experimental import pallas as pl
from jax.experimental.pallas import tpu as pltpu
from jax.experimental.pallas import tpu_sc as plsc

BATCH = 4096
SEQ = 200
HIDDEN = 64
VOCAB = 1000000
N = BATCH * SEQ
HEAD = 128
TAIL = SEQ - HEAD

_info = plsc.get_sparse_core_info()
NC, NS = _info.num_cores, _info.num_subcores
NW = NC * NS

ROWS_PER_W = BATCH // NW
NBUF = 4

_mesh = plsc.VectorSubcoreMesh(core_axis_name="c", subcore_axis_name="s")


@functools.partial(
    pl.kernel,
    out_type=jax.ShapeDtypeStruct((2 * BATCH * HEAD,), jnp.int32),
    mesh=_mesh,
    scratch_types=[
        pltpu.VMEM((ROWS_PER_W, HEAD), jnp.int32),
        pltpu.VMEM((ROWS_PER_W, HEAD), jnp.int32),
        pltpu.SemaphoreType.DMA,
        pltpu.SemaphoreType.DMA,
    ],
)
def _repack_ids(ids_hbm, out_hbm, hbuf, tbuf, sem_in, sem_out):
    wid = lax.axis_index("s") * NC + lax.axis_index("c")
    row0 = wid * ROWS_PER_W
    pltpu.make_async_copy(
        ids_hbm.at[pl.ds(row0, ROWS_PER_W), pl.ds(0, HEAD)], hbuf,
        sem_in).start()
    pltpu.make_async_copy(
        ids_hbm.at[pl.ds(row0, ROWS_PER_W), pl.ds(HEAD, HEAD)], tbuf,
        sem_in).start()
    pltpu.make_async_copy(
        ids_hbm.at[pl.ds(row0, ROWS_PER_W), pl.ds(0, HEAD)], hbuf,
        sem_in).wait()
    pltpu.make_async_copy(
        ids_hbm.at[pl.ds(row0, ROWS_PER_W), pl.ds(HEAD, HEAD)], tbuf,
        sem_in).wait()
    cps = []
    for r in range(ROWS_PER_W):
        cps.append(pltpu.make_async_copy(
            hbuf.at[r], out_hbm.at[pl.ds((row0 + r) * HEAD, HEAD)],
            sem_out))
        cps.append(pltpu.make_async_copy(
            tbuf.at[r],
            out_hbm.at[pl.ds((BATCH + row0 + r) * HEAD, HEAD)], sem_out))
    for c in cps:
        c.start()
    for c in cps:
        c.wait()


@functools.partial(
    pl.kernel,
    out_type=jax.ShapeDtypeStruct((BATCH, SEQ, 2 * HIDDEN), jnp.float32),
    mesh=_mesh,
    scratch_types=[
        pltpu.VMEM((NBUF, 2 * HEAD), jnp.int32),
        pltpu.VMEM((NBUF, SEQ, HIDDEN), jnp.float32),
        pltpu.SemaphoreType.DMA((NBUF,)),
        pltpu.SemaphoreType.DMA((NBUF,)),
        pltpu.SemaphoreType.DMA((NBUF,)),
    ],
    compiler_params=pltpu.CompilerParams(use_tc_tiling_on_sc=False),
)
def _gather_kernel(ids_hbm, tab_hbm, out_hbm, idx_v, rows_v, sem_i, sem_g,
                   sem_o):
    wid = lax.axis_index("s") * NC + lax.axis_index("c")
    base = wid * ROWS_PER_W

    def idx_copies(c, slot):
        return [
            pltpu.make_async_copy(
                ids_hbm.at[pl.ds((base + c) * HEAD, HEAD)],
                idx_v.at[slot, pl.ds(0, HEAD)], sem_i.at[slot]),
            pltpu.make_async_copy(
                ids_hbm.at[pl.ds((BATCH + base + c) * HEAD, HEAD)],
                idx_v.at[slot, pl.ds(HEAD, HEAD)], sem_i.at[slot]),
        ]

    def gathers(c, slot):
        return [
            pltpu.make_async_copy(
                tab_hbm.at[idx_v.at[slot, pl.ds(0, HEAD)]],
                rows_v.at[slot, pl.ds(0, HEAD), :], sem_g.at[slot]),
            pltpu.make_async_copy(
                tab_hbm.at[idx_v.at[slot, pl.ds(HEAD, TAIL)]],
                rows_v.at[slot, pl.ds(HEAD, TAIL), :], sem_g.at[slot]),
        ]

    def out_copy(c, slot):
        return pltpu.make_async_copy(
            rows_v.at[slot],
            out_hbm.at[base + c, :, pl.ds(0, HIDDEN)], sem_o.at[slot])

    for b in range(NBUF):
        for cp in idx_copies(b, b):
            cp.start()

    def group(g, carry):
        for b in range(NBUF):
            c = g * NBUF + b
            for cp in idx_copies(c, b):
                cp.wait()
            pl.when(g > 0)(lambda b=b: out_copy(0, b).wait())
            for cp in gathers(c, b):
                cp.start()
            bp = (b - 1) % NBUF

            def finish(c=c, b=b, bp=bp):
                for cp in gathers(c - 1, bp):
                    cp.wait()
                out_copy(c - 1, bp).start()

                def prefetch(c=c, bp=bp):
                    for cp in idx_copies(c + NBUF - 1, bp):
                        cp.start()

                pl.when(c + NBUF - 1 < ROWS_PER_W)(prefetch)

            if b == 0:
                pl.when(g > 0)(finish)
            else:
                finish()
        return carry

    lax.fori_loop(0, ROWS_PER_W // NBUF, group, 0, unroll=False)

    bl = (ROWS_PER_W - 1) % NBUF
    for cp in gathers(ROWS_PER_W - 1, bl):
        cp.wait()
    out_copy(ROWS_PER_W - 1, bl).start()
    for b in range(NBUF):
        out_copy(0, b).wait()


def kernel(input_ids, word_embeddings):
    ids2 = jnp.pad(input_ids.astype(jnp.int32) * 2,
                   ((0, 0), (0, 2 * HEAD - SEQ)))
    packed = _repack_ids(ids2)
    tab2 = jnp.pad(word_embeddings, ((0, 0), (0, HIDDEN))
                   ).reshape(2 * VOCAB, HIDDEN)
    out1 = _gather_kernel(packed, tab2)
    return out1[:, :, :HIDDEN]


# speedup vs baseline: 1.4294x; 1.0038x over previous
import functools

import jax
import jax.numpy as jnp
from jax import lax
from jax.experimental import pallas as pl
from jax.experimental.pallas import tpu as pltpu
from jax.experimental.pallas import tpu_sc as plsc

BATCH = 4096
SEQ = 200
HIDDEN = 64
VOCAB = 1000000
N = BATCH * SEQ
HEAD = 128
TAIL = SEQ - HEAD

_info = plsc.get_sparse_core_info()
NC, NS = _info.num_cores, _info.num_subcores
NW = NC * NS

ROWS_PER_W = BATCH // NW
NBUF = 8

_mesh = plsc.VectorSubcoreMesh(core_axis_name="c", subcore_axis_name="s")


@functools.partial(
    pl.kernel,
    out_type=jax.ShapeDtypeStruct((2 * BATCH * HEAD,), jnp.int32),
    mesh=_mesh,
    scratch_types=[
        pltpu.VMEM((ROWS_PER_W, HEAD), jnp.int32),
        pltpu.VMEM((ROWS_PER_W, HEAD), jnp.int32),
        pltpu.SemaphoreType.DMA,
        pltpu.SemaphoreType.DMA,
    ],
)
def _repack_ids(ids_hbm, out_hbm, hbuf, tbuf, sem_in, sem_out):
    wid = lax.axis_index("s") * NC + lax.axis_index("c")
    row0 = wid * ROWS_PER_W
    pltpu.make_async_copy(
        ids_hbm.at[pl.ds(row0, ROWS_PER_W), pl.ds(0, HEAD)], hbuf,
        sem_in).start()
    pltpu.make_async_copy(
        ids_hbm.at[pl.ds(row0, ROWS_PER_W), pl.ds(HEAD, HEAD)], tbuf,
        sem_in).start()
    pltpu.make_async_copy(
        ids_hbm.at[pl.ds(row0, ROWS_PER_W), pl.ds(0, HEAD)], hbuf,
        sem_in).wait()
    pltpu.make_async_copy(
        ids_hbm.at[pl.ds(row0, ROWS_PER_W), pl.ds(HEAD, HEAD)], tbuf,
        sem_in).wait()
    cps = []
    for r in range(ROWS_PER_W):
        cps.append(pltpu.make_async_copy(
            hbuf.at[r], out_hbm.at[pl.ds((row0 + r) * HEAD, HEAD)],
            sem_out))
        cps.append(pltpu.make_async_copy(
            tbuf.at[r],
            out_hbm.at[pl.ds((BATCH + row0 + r) * HEAD, HEAD)], sem_out))
    for c in cps:
        c.start()
    for c in cps:
        c.wait()


@functools.partial(
    pl.kernel,
    out_type=jax.ShapeDtypeStruct((BATCH, SEQ, 2 * HIDDEN), jnp.float32),
    mesh=_mesh,
    scratch_types=[
        pltpu.VMEM((NBUF, 2 * HEAD), jnp.int32),
        pltpu.VMEM((NBUF, SEQ, HIDDEN), jnp.float32),
        pltpu.SemaphoreType.DMA((NBUF,)),
        pltpu.SemaphoreType.DMA((NBUF,)),
        pltpu.SemaphoreType.DMA((NBUF,)),
    ],
    compiler_params=pltpu.CompilerParams(use_tc_tiling_on_sc=False),
)
def _gather_kernel(ids_hbm, tab_hbm, out_hbm, idx_v, rows_v, sem_i, sem_g,
                   sem_o):
    wid = lax.axis_index("s") * NC + lax.axis_index("c")
    base = wid * ROWS_PER_W

    def idx_copies(c, slot):
        return [
            pltpu.make_async_copy(
                ids_hbm.at[pl.ds((base + c) * HEAD, HEAD)],
                idx_v.at[slot, pl.ds(0, HEAD)], sem_i.at[slot]),
            pltpu.make_async_copy(
                ids_hbm.at[pl.ds((BATCH + base + c) * HEAD, HEAD)],
                idx_v.at[slot, pl.ds(HEAD, HEAD)], sem_i.at[slot]),
        ]

    def gathers(c, slot):
        return [
            pltpu.make_async_copy(
                tab_hbm.at[idx_v.at[slot, pl.ds(0, HEAD)]],
                rows_v.at[slot, pl.ds(0, HEAD), :], sem_g.at[slot]),
            pltpu.make_async_copy(
                tab_hbm.at[idx_v.at[slot, pl.ds(HEAD, TAIL)]],
                rows_v.at[slot, pl.ds(HEAD, TAIL), :], sem_g.at[slot]),
        ]

    def out_copy(c, slot):
        return pltpu.make_async_copy(
            rows_v.at[slot],
            out_hbm.at[base + c, :, pl.ds(0, HIDDEN)], sem_o.at[slot])

    for b in range(NBUF):
        for cp in idx_copies(b, b):
            cp.start()

    def group(g, carry):
        for b in range(NBUF):
            c = g * NBUF + b
            for cp in idx_copies(c, b):
                cp.wait()
            pl.when(g > 0)(lambda b=b: out_copy(0, b).wait())
            for cp in gathers(c, b):
                cp.start()
            bp = (b - 1) % NBUF

            def finish(c=c, b=b, bp=bp):
                for cp in gathers(c - 1, bp):
                    cp.wait()
                out_copy(c - 1, bp).start()

                def prefetch(c=c, bp=bp):
                    for cp in idx_copies(c + NBUF - 1, bp):
                        cp.start()

                pl.when(c + NBUF - 1 < ROWS_PER_W)(prefetch)

            if b == 0:
                pl.when(g > 0)(finish)
            else:
                finish()
        return carry

    lax.fori_loop(0, ROWS_PER_W // NBUF, group, 0, unroll=False)

    bl = (ROWS_PER_W - 1) % NBUF
    for cp in gathers(ROWS_PER_W - 1, bl):
        cp.wait()
    out_copy(ROWS_PER_W - 1, bl).start()
    for b in range(NBUF):
        out_copy(0, b).wait()


def kernel(input_ids, word_embeddings):
    ids2 = jnp.pad(input_ids.astype(jnp.int32) * 2,
                   ((0, 0), (0, 2 * HEAD - SEQ)))
    packed = _repack_ids(ids2)
    tab2 = jnp.pad(word_embeddings, ((0, 0), (0, HIDDEN))
                   ).reshape(2 * VOCAB, HIDDEN)
    out1 = _gather_kernel(packed, tab2)
    return out1[:, :, :HIDDEN]
